# baseline (speedup 1.0000x reference)
import jax
import jax.numpy as jnp
from jax import lax
from jax.experimental import pallas as pl
from jax.experimental.pallas import tpu as pltpu
from jax.experimental.pallas import tpu_sc as plsc

B = 16384
D = 64
L = 16

_info = plsc.get_sparse_core_info()
NC = _info.num_cores
NS = _info.num_subcores
NW = NC * NS
CHUNK = B // NW
GCH = 128
NG = CHUNK // GCH


def _body(uid_hbm, iid_hbm, nid_hbm, utab_hbm, itab_hbm,
          pos_hbm, neg_hbm,
          idx_u, idx_p, idx_n, rows_u, rows_p, rows_n,
          out_p, out_n, sem_idx, sem_g):
    wid = lax.axis_index("s") * NC + lax.axis_index("c")
    base = wid * CHUNK

    idx_dmas = []
    for j in range(NG):
        off = base + j * GCH
        idx_dmas.append(
            pltpu.async_copy(uid_hbm.at[pl.ds(off, GCH)], idx_u.at[j], sem_idx))
        idx_dmas.append(
            pltpu.async_copy(iid_hbm.at[pl.ds(off, GCH)], idx_p.at[j], sem_idx))
        idx_dmas.append(
            pltpu.async_copy(nid_hbm.at[pl.ds(off, GCH)], idx_n.at[j], sem_idx))
    for h in idx_dmas:
        h.wait()

    g_dmas = []
    for j in range(NG):
        g_dmas.append(
            pltpu.async_copy(utab_hbm.at[idx_u.at[j]],
                             rows_u.at[pl.ds(j * GCH, GCH)], sem_g))
        g_dmas.append(
            pltpu.async_copy(itab_hbm.at[idx_p.at[j]],
                             rows_p.at[pl.ds(j * GCH, GCH)], sem_g))
        g_dmas.append(
            pltpu.async_copy(itab_hbm.at[idx_n.at[j]],
                             rows_n.at[pl.ds(j * GCH, GCH)], sem_g))
    for h in g_dmas:
        h.wait()

    lanes = lax.iota(jnp.int32, L)

    def group(g, _):
        accP = jnp.zeros((L,), jnp.float32)
        accN = jnp.zeros((L,), jnp.float32)
        for i in range(L):
            t = g * L + i
            sp = jnp.zeros((L,), jnp.float32)
            sn = jnp.zeros((L,), jnp.float32)
            for k in range(D // L):
                u = rows_u[t, pl.ds(k * L, L)]
                p = rows_p[t, pl.ds(k * L, L)]
                n = rows_n[t, pl.ds(k * L, L)]
                sp = sp + u * p
                sn = sn + u * n
            accP = jnp.where(lanes == i, jnp.sum(sp), accP)
            accN = jnp.where(lanes == i, jnp.sum(sn), accN)
        out_p[pl.ds(g * L, L)] = accP
        out_n[pl.ds(g * L, L)] = accN
        return _

    lax.fori_loop(0, CHUNK // L, group, None)

    pltpu.sync_copy(out_p, pos_hbm.at[pl.ds(base, CHUNK)])
    pltpu.sync_copy(out_n, neg_hbm.at[pl.ds(base, CHUNK)])


def kernel(user_id, item_id, neg_item_id, user_table, item_table):
    mesh = plsc.VectorSubcoreMesh(core_axis_name="c", subcore_axis_name="s")
    f = pl.kernel(
        _body,
        mesh=mesh,
        compiler_params=pltpu.CompilerParams(
            needs_layout_passes=False,
            use_tc_tiling_on_sc=False,
        ),
        out_type=(
            jax.ShapeDtypeStruct((B,), jnp.float32),
            jax.ShapeDtypeStruct((B,), jnp.float32),
        ),
        scratch_types=[
            pltpu.VMEM((NG, GCH), jnp.int32),
            pltpu.VMEM((NG, GCH), jnp.int32),
            pltpu.VMEM((NG, GCH), jnp.int32),
            pltpu.VMEM((CHUNK, D), jnp.float32),
            pltpu.VMEM((CHUNK, D), jnp.float32),
            pltpu.VMEM((CHUNK, D), jnp.float32),
            pltpu.VMEM((CHUNK,), jnp.float32),
            pltpu.VMEM((CHUNK,), jnp.float32),
            pltpu.SemaphoreType.DMA,
            pltpu.SemaphoreType.DMA,
        ],
    )
    return f(user_id.astype(jnp.int32), item_id.astype(jnp.int32),
             neg_item_id.astype(jnp.int32), user_table, item_table)
